# baseline (speedup 1.0000x reference)
import functools

import jax
import jax.numpy as jnp
from jax.experimental import pallas as pl
from jax.experimental.pallas import tpu as pltpu


def _fused_kernel(x_ref, w1_ref, b1_ref, w2_ref, b2_ref, wf_ref, bf_ref,
                  col_ref, wc1_ref, bc1_ref, wc2_ref, bc2_ref,
                  wfc_ref, bfc_ref, wo_ref, bo_ref, out_ref, acc_ref,
                  *, n_nodes, n_cols, d24):
    j = pl.program_id(1)
    nb = pl.num_programs(1)

    @pl.when(j == 0)
    def _init():
        acc_ref[...] = jnp.zeros_like(acc_ref)

    x = x_ref[0]
    h1t = jnp.maximum(
        jax.lax.dot_general(w1_ref[...], x, (((0,), (1,)), ((), ())),
                            preferred_element_type=jnp.float32)
        + b1_ref[...], 0.0)
    h2t = jnp.maximum(
        jax.lax.dot_general(w2_ref[...], h1t, (((0,), (0,)), ((), ())),
                            preferred_element_type=jnp.float32)
        + b2_ref[...], 0.0)
    acc_ref[...] += jnp.sum(h2t, axis=1, keepdims=True)

    @pl.when(j == nb - 1)
    def _tail():
        node_sum = jax.lax.dot_general(
            acc_ref[...], wf_ref[...], (((0,), (0,)), ((), ())),
            preferred_element_type=jnp.float32)
        node_avg = node_sum / float(n_nodes) + bf_ref[...]

        cb = col_ref[0]
        hc = jnp.maximum(
            jnp.dot(cb, wc1_ref[...], preferred_element_type=jnp.float32)
            + bc1_ref[...], 0.0)
        vc = jnp.dot(hc, wc2_ref[...], preferred_element_type=jnp.float32)

        m = cb.shape[0]
        rows = jax.lax.broadcasted_iota(jnp.int32, (m, d24), 0)
        cols = jax.lax.broadcasted_iota(jnp.int32, (m, d24), 1)
        selT = (rows % d24 == cols).astype(jnp.float32)
        col_sum = jax.lax.dot_general(
            vc, selT, (((0,), (0,)), ((), ())), preferred_element_type=jnp.float32)
        col_avg = col_sum / float(n_cols) + bc2_ref[...]

        z = jnp.maximum(
            node_avg * wfc_ref[0:1, :]
            + jnp.dot(col_avg, wfc_ref[1:, :], preferred_element_type=jnp.float32)
            + bfc_ref[...], 0.0)
        out_ref[0] = jnp.dot(
            z, wo_ref[...], preferred_element_type=jnp.float32) + bo_ref[...]


def kernel(node_features, col_features, edge_index, W1, b1, W2, b2, Wf, bf,
           Wc1, bc1, Wc2, bc2, Wfc, bfc, Wo, bo):
    B, N, F = node_features.shape
    _, C, D24, CF = col_features.shape
    H = W1.shape[1]

    blk = N
    for cand in (2000, 1000, 200, 8):
        if N % cand == 0 and cand % 8 == 0:
            blk = cand
            break
    nb = N // blk

    b1r = b1.reshape(H, 1)
    b2r = b2.reshape(H, 1)
    bc1r = bc1.reshape(1, H)
    bc2r = bc2.reshape(1, 1)
    bfr = bf.reshape(1, 1)
    bfcr = bfc.reshape(1, Wfc.shape[1])
    bor = bo.reshape(1, 1)
    colr = col_features.reshape(B, C * D24, CF)

    out = pl.pallas_call(
        functools.partial(_fused_kernel, n_nodes=N, n_cols=C, d24=D24),
        grid=(B, nb),
        in_specs=[
            pl.BlockSpec((1, blk, F), lambda b, j: (b, j, 0)),
            pl.BlockSpec((F, H), lambda b, j: (0, 0)),
            pl.BlockSpec((H, 1), lambda b, j: (0, 0)),
            pl.BlockSpec((H, H), lambda b, j: (0, 0)),
            pl.BlockSpec((H, 1), lambda b, j: (0, 0)),
            pl.BlockSpec((H, 1), lambda b, j: (0, 0)),
            pl.BlockSpec((1, 1), lambda b, j: (0, 0)),
            pl.BlockSpec((1, C * D24, CF), lambda b, j: (b, 0, 0)),
            pl.BlockSpec((CF, H), lambda b, j: (0, 0)),
            pl.BlockSpec((1, H), lambda b, j: (0, 0)),
            pl.BlockSpec((H, 1), lambda b, j: (0, 0)),
            pl.BlockSpec((1, 1), lambda b, j: (0, 0)),
            pl.BlockSpec(Wfc.shape, lambda b, j: (0, 0)),
            pl.BlockSpec((1, Wfc.shape[1]), lambda b, j: (0, 0)),
            pl.BlockSpec((Wfc.shape[1], 1), lambda b, j: (0, 0)),
            pl.BlockSpec((1, 1), lambda b, j: (0, 0)),
        ],
        out_specs=pl.BlockSpec((1, 1, 1), lambda b, j: (b, 0, 0)),
        out_shape=jax.ShapeDtypeStruct((B, 1, 1), jnp.float32),
        scratch_shapes=[pltpu.VMEM((H, 1), jnp.float32)],
    )(node_features, W1, b1r, W2, b2r, Wf, bfr,
      colr, Wc1, bc1r, Wc2, bc2r, Wfc, bfcr, Wo, bor)

    return out.reshape(B, 1)


# speedup vs baseline: 1.2778x; 1.2778x over previous
import functools

import jax
import jax.numpy as jnp
from jax.experimental import pallas as pl
from jax.experimental.pallas import tpu as pltpu


def _fused_kernel(x_ref, w1_ref, b1_ref, w2_ref, b2_ref, wf_ref, bf_ref,
                  col_ref, wc1_ref, bc1_ref, wc2_ref, bc2_ref,
                  wfc_ref, bfc_ref, wo_ref, bo_ref, out_ref, acc_ref,
                  *, n_nodes, n_cols, d24):
    j = pl.program_id(1)
    nb = pl.num_programs(1)

    @pl.when(j == 0)
    def _init():
        acc_ref[...] = jnp.zeros_like(acc_ref)

    x = x_ref[0]
    h1t = jnp.maximum(
        jax.lax.dot_general(w1_ref[...], x, (((0,), (1,)), ((), ())),
                            preferred_element_type=jnp.float32)
        + b1_ref[...], 0.0)
    h2t = jnp.maximum(
        jax.lax.dot_general(w2_ref[...], h1t, (((0,), (0,)), ((), ())),
                            preferred_element_type=jnp.float32)
        + b2_ref[...], 0.0)
    acc_ref[...] += jnp.sum(h2t, axis=1, keepdims=True)

    @pl.when(j == nb - 1)
    def _tail():
        node_sum = jax.lax.dot_general(
            acc_ref[...], wf_ref[...], (((0,), (0,)), ((), ())),
            preferred_element_type=jnp.float32)
        node_avg = node_sum / float(n_nodes) + bf_ref[...]

        cb = col_ref[0]
        hc = jnp.maximum(
            jnp.dot(cb, wc1_ref[...], preferred_element_type=jnp.float32)
            + bc1_ref[...], 0.0)
        vc = jnp.dot(hc, wc2_ref[...], preferred_element_type=jnp.float32)

        m = cb.shape[0]
        rows = jax.lax.broadcasted_iota(jnp.int32, (m, d24), 0)
        cols = jax.lax.broadcasted_iota(jnp.int32, (m, d24), 1)
        selT = (rows % d24 == cols).astype(jnp.float32)
        col_sum = jax.lax.dot_general(
            vc, selT, (((0,), (0,)), ((), ())), preferred_element_type=jnp.float32)
        col_avg = col_sum / float(n_cols) + bc2_ref[...]

        z = jnp.maximum(
            node_avg * wfc_ref[0:1, :]
            + jnp.dot(col_avg, wfc_ref[1:, :], preferred_element_type=jnp.float32)
            + bfc_ref[...], 0.0)
        out_ref[0] = jnp.dot(
            z, wo_ref[...], preferred_element_type=jnp.float32) + bo_ref[...]


def kernel(node_features, col_features, edge_index, W1, b1, W2, b2, Wf, bf,
           Wc1, bc1, Wc2, bc2, Wfc, bfc, Wo, bo):
    B, N, F = node_features.shape
    _, C, D24, CF = col_features.shape
    H = W1.shape[1]

    blk = N
    for cand in (10000, 2000, 1000, 200, 8):
        if N % cand == 0 and cand % 8 == 0:
            blk = cand
            break
    nb = N // blk

    b1r = b1.reshape(H, 1)
    b2r = b2.reshape(H, 1)
    bc1r = bc1.reshape(1, H)
    bc2r = bc2.reshape(1, 1)
    bfr = bf.reshape(1, 1)
    bfcr = bfc.reshape(1, Wfc.shape[1])
    bor = bo.reshape(1, 1)
    colr = col_features.reshape(B, C * D24, CF)

    out = pl.pallas_call(
        functools.partial(_fused_kernel, n_nodes=N, n_cols=C, d24=D24),
        grid=(B, nb),
        in_specs=[
            pl.BlockSpec((1, blk, F), lambda b, j: (b, j, 0)),
            pl.BlockSpec((F, H), lambda b, j: (0, 0)),
            pl.BlockSpec((H, 1), lambda b, j: (0, 0)),
            pl.BlockSpec((H, H), lambda b, j: (0, 0)),
            pl.BlockSpec((H, 1), lambda b, j: (0, 0)),
            pl.BlockSpec((H, 1), lambda b, j: (0, 0)),
            pl.BlockSpec((1, 1), lambda b, j: (0, 0)),
            pl.BlockSpec((1, C * D24, CF), lambda b, j: (b, 0, 0)),
            pl.BlockSpec((CF, H), lambda b, j: (0, 0)),
            pl.BlockSpec((1, H), lambda b, j: (0, 0)),
            pl.BlockSpec((H, 1), lambda b, j: (0, 0)),
            pl.BlockSpec((1, 1), lambda b, j: (0, 0)),
            pl.BlockSpec(Wfc.shape, lambda b, j: (0, 0)),
            pl.BlockSpec((1, Wfc.shape[1]), lambda b, j: (0, 0)),
            pl.BlockSpec((Wfc.shape[1], 1), lambda b, j: (0, 0)),
            pl.BlockSpec((1, 1), lambda b, j: (0, 0)),
        ],
        out_specs=pl.BlockSpec((1, 1, 1), lambda b, j: (b, 0, 0)),
        out_shape=jax.ShapeDtypeStruct((B, 1, 1), jnp.float32),
        scratch_shapes=[pltpu.VMEM((H, 1), jnp.float32)],
    )(node_features, W1, b1r, W2, b2r, Wf, bfr,
      colr, Wc1, bc1r, Wc2, bc2r, Wfc, bfcr, Wo, bor)

    return out.reshape(B, 1)
